# baseline (speedup 1.0000x reference)
import functools

import jax
import jax.numpy as jnp
from jax import lax
from jax.experimental import pallas as pl
from jax.experimental.pallas import tpu as pltpu
from jax.experimental.pallas import tpu_sc as plsc

NC = 2
NS = 16
NW = NC * NS
G = 128
CH = 256
NSLOT = 2


def _build(n, d):
  ipw = n // NW
  nch = ipw // CH
  mesh = plsc.VectorSubcoreMesh(core_axis_name="c", subcore_axis_name="s")

  @functools.partial(
      pl.kernel,
      out_type=jax.ShapeDtypeStruct((n, d), jnp.float32),
      mesh=mesh,
      compiler_params=pltpu.CompilerParams(use_tc_tiling_on_sc=False),
      scratch_types=[
          pltpu.VMEM((ipw,), jnp.int32),
          pltpu.VMEM((NSLOT, CH, d), jnp.float32),
      ] + [pltpu.SemaphoreType.DMA] * (NSLOT + 1),
  )
  def k(idx_hbm, table_hbm, out_hbm, idx_v, rows_v, *sems):
    gsem = sems[:NSLOT]
    wsem = sems[NSLOT]
    wid = lax.axis_index("s") * NC + lax.axis_index("c")
    base = wid * ipw
    pltpu.sync_copy(idx_hbm.at[pl.ds(base, ipw)], idx_v)

    def fire(c, s):
      for j in range(CH // G):
        pltpu.async_copy(
            table_hbm.at[idx_v.at[pl.ds(c * CH + j * G, G)]],
            rows_v.at[s, pl.ds(j * G, G)], gsem[s])

    for s in range(NSLOT):
      fire(s, s)

    def body(t, carry):
      for s in range(NSLOT):
        c = t * NSLOT + s
        pltpu.make_async_copy(
            out_hbm.at[pl.ds(0, CH)], rows_v.at[s], gsem[s]).wait()
        fire(c, s)
      return carry

    lax.fori_loop(1, nch // NSLOT, body, 0)

    for s in range(NSLOT):
      pltpu.make_async_copy(
          out_hbm.at[pl.ds(0, CH)], rows_v.at[s], gsem[s]).wait()

    for c in range(0, nch, NSLOT):
      for s in range(NSLOT):
        pltpu.async_copy(
            rows_v.at[s], out_hbm.at[pl.ds(base + (c + s) * CH, CH)], wsem)
    pltpu.make_async_copy(
        out_hbm.at[pl.ds(0, nch * CH)],
        out_hbm.at[pl.ds(0, nch * CH)], wsem).wait()

  return k


def kernel(indices, table):
  n = indices.size
  d = table.shape[1]
  out = _build(n, d)(indices.reshape(n), table)
  return out.reshape(*indices.shape, d)


# speedup vs baseline: 1.0116x; 1.0116x over previous
import functools

import jax
import jax.numpy as jnp
from jax import lax
from jax.experimental import pallas as pl
from jax.experimental.pallas import tpu as pltpu
from jax.experimental.pallas import tpu_sc as plsc

NC = 2
NS = 16
NW = NC * NS
G = 128
CH = 256
NSLOT = 5


def _build(n, d):
  ipw = n // NW
  nch = ipw // CH
  mesh = plsc.VectorSubcoreMesh(core_axis_name="c", subcore_axis_name="s")

  @functools.partial(
      pl.kernel,
      out_type=jax.ShapeDtypeStruct((n, d), jnp.float32),
      mesh=mesh,
      compiler_params=pltpu.CompilerParams(use_tc_tiling_on_sc=False),
      scratch_types=[
          pltpu.VMEM((ipw,), jnp.int32),
          pltpu.VMEM((NSLOT, CH, d), jnp.float32),
      ] + [pltpu.SemaphoreType.DMA] * (2 * NSLOT),
  )
  def k(idx_hbm, table_hbm, out_hbm, idx_v, rows_v, *sems):
    gsem = sems[:NSLOT]
    wsem = sems[NSLOT:]
    wid = lax.axis_index("s") * NC + lax.axis_index("c")
    base = wid * ipw
    pltpu.sync_copy(idx_hbm.at[pl.ds(base, ipw)], idx_v)

    def fire(c, s):
      for j in range(CH // G):
        pltpu.async_copy(
            table_hbm.at[idx_v.at[pl.ds(c * CH + j * G, G)]],
            rows_v.at[s, pl.ds(j * G, G)], gsem[s])

    def drain(dc, s_d):
      pltpu.make_async_copy(
          out_hbm.at[pl.ds(0, CH)], rows_v.at[s_d], gsem[s_d]).wait()
      pltpu.async_copy(
          rows_v.at[s_d], out_hbm.at[pl.ds(base + dc * CH, CH)], wsem[s_d])

    for s in range(NSLOT):
      fire(s, s)
      if s == NSLOT - 1:
        drain(0, 0)

    def body(t, carry):
      for s in range(NSLOT):
        c = t * NSLOT + s
        pltpu.make_async_copy(
            rows_v.at[s], out_hbm.at[pl.ds(0, CH)], wsem[s]).wait()
        fire(c, s)
        drain(c - (NSLOT - 1), (s + 1) % NSLOT)
      return carry

    lax.fori_loop(1, nch // NSLOT, body, 0)

    for q in range(NSLOT - 1):
      dc = nch - (NSLOT - 1) + q
      drain(dc, dc % NSLOT)
    for s in range(NSLOT):
      pltpu.make_async_copy(
          rows_v.at[s], out_hbm.at[pl.ds(0, CH)], wsem[s]).wait()

  return k


def kernel(indices, table):
  n = indices.size
  d = table.shape[1]
  out = _build(n, d)(indices.reshape(n), table)
  return out.reshape(*indices.shape, d)
